# Initial kernel scaffold; baseline (speedup 1.0000x reference)
import functools

import jax
import jax.numpy as jnp
from jax import lax
from jax.experimental import pallas as pl
from jax.experimental.pallas import tpu as pltpu
from jax.experimental.pallas import tpu_sc as plsc

N = 10000
H = 8
D = 128
D_IN = 128
E_RAW = 320000
E_TOT = E_RAW + N
NC, NS = 2, 16
NW = NC * NS
BLK = 512
BPW = 21
E_PAD = NW * BPW * BLK
ROWS = E_PAD // 128
RPW = ROWS // NW
NPT = N // NS
BN = 1000

_mesh = plsc.VectorSubcoreMesh(core_axis_name="c", subcore_axis_name="s")
f32 = jnp.float32
i32 = jnp.int32


def _tc_proj_body(x_ref, w_ref, am_ref, bm_ref, ht_ref, as_ref, ad_ref):
    xb = x_ref[...]
    hb = jnp.dot(xb, w_ref[...], preferred_element_type=f32)
    for j in range(H):
        ht_ref[j] = hb[:, j * D:(j + 1) * D]
    as_ref[...] = jnp.dot(hb, am_ref[...], preferred_element_type=f32)
    ad_ref[...] = jnp.dot(hb, bm_ref[...], preferred_element_type=f32)


def _tc_proj(x, w, am, bm):
    return pl.pallas_call(
        _tc_proj_body,
        grid=(N // BN,),
        in_specs=[pl.BlockSpec((BN, D_IN), lambda i: (i, 0)),
                  pl.BlockSpec((D_IN, H * D), lambda i: (0, 0)),
                  pl.BlockSpec((H * D, 16), lambda i: (0, 0)),
                  pl.BlockSpec((H * D, 16), lambda i: (0, 0))],
        out_specs=[pl.BlockSpec((H, BN, D), lambda i: (0, i, 0)),
                   pl.BlockSpec((BN, 16), lambda i: (i, 0)),
                   pl.BlockSpec((BN, 16), lambda i: (i, 0))],
        out_shape=[jax.ShapeDtypeStruct((H, N, D), f32),
                   jax.ShapeDtypeStruct((N, 16), f32),
                   jax.ShapeDtypeStruct((N, 16), f32)],
    )(x, w, am, bm)


def _tc_mid_body(o_ref, b_ref, w_ref, am_ref, bm_ref, ht_ref, as_ref, ad_ref):
    hin = jnp.maximum((o_ref[0] + o_ref[1]) * (1.0 / H) + b_ref[...], 0.0)
    hb = jnp.dot(hin, w_ref[...], preferred_element_type=f32)
    for j in range(H):
        ht_ref[j] = hb[:, j * D:(j + 1) * D]
    as_ref[...] = jnp.dot(hb, am_ref[...], preferred_element_type=f32)
    ad_ref[...] = jnp.dot(hb, bm_ref[...], preferred_element_type=f32)


def _tc_mid(outp, b, w, am, bm):
    return pl.pallas_call(
        _tc_mid_body,
        grid=(N // BN,),
        in_specs=[pl.BlockSpec((NC, BN, D), lambda i: (0, i, 0)),
                  pl.BlockSpec((1, D), lambda i: (0, 0)),
                  pl.BlockSpec((D, H * D), lambda i: (0, 0)),
                  pl.BlockSpec((H * D, 16), lambda i: (0, 0)),
                  pl.BlockSpec((H * D, 16), lambda i: (0, 0))],
        out_specs=[pl.BlockSpec((H, BN, D), lambda i: (0, i, 0)),
                   pl.BlockSpec((BN, 16), lambda i: (i, 0)),
                   pl.BlockSpec((BN, 16), lambda i: (i, 0))],
        out_shape=[jax.ShapeDtypeStruct((H, N, D), f32),
                   jax.ShapeDtypeStruct((N, 16), f32),
                   jax.ShapeDtypeStruct((N, 16), f32)],
    )(outp, b, w, am, bm)


def _tc_fin_body(o_ref, b_ref, out_ref):
    out_ref[...] = (o_ref[0] + o_ref[1]) * (1.0 / H) + b_ref[...]


def _tc_fin(outp, b):
    return pl.pallas_call(
        _tc_fin_body,
        grid=(N // BN,),
        in_specs=[pl.BlockSpec((NC, BN, D), lambda i: (0, i, 0)),
                  pl.BlockSpec((1, D), lambda i: (0, 0))],
        out_specs=pl.BlockSpec((BN, D), lambda i: (i, 0)),
        out_shape=jax.ShapeDtypeStruct((N, D), f32),
    )(outp, b)


@functools.partial(
    pl.kernel, mesh=_mesh,
    out_type=(jax.ShapeDtypeStruct((E_PAD, 16), f32),
              jax.ShapeDtypeStruct((NC, N, 16), f32)),
    scratch_types=[pltpu.VMEM((4, 128), i32),
                   pltpu.VMEM((4, 128), i32),
                   pltpu.VMEM((BLK, 16), f32),
                   pltpu.VMEM((BLK, 16), f32),
                   pltpu.VMEM((BLK, 16), f32),
                   pltpu.VMEM_SHARED((N, 16), f32)])
def _phase_a(srcA_hbm, dstP_hbm, asrc_hbm, adst_hbm, p_hbm, dpart_hbm,
             isrc, idst, gs, gd, pbuf, dacc):
    c = lax.axis_index("c")
    s = lax.axis_index("s")
    wid = c * NS + s
    nbase = s * NPT

    @pl.loop(0, BLK)
    def _zero(i):
        pbuf[i, :] = jnp.zeros((16,), f32)

    pltpu.sync_copy(pbuf.at[pl.ds(0, 512)], dacc.at[pl.ds(nbase, 512)])
    pltpu.sync_copy(pbuf.at[pl.ds(0, NPT - 512)],
                    dacc.at[pl.ds(nbase + 512, NPT - 512)])
    plsc.subcore_barrier()

    row0 = wid * RPW

    @pl.loop(0, BPW)
    def _block(b):
        r = row0 + b * 4
        ebase = r * 128
        pltpu.sync_copy(srcA_hbm.at[pl.ds(r, 4)], isrc)
        pltpu.sync_copy(dstP_hbm.at[pl.ds(r, 4)], idst)
        for m in range(4):
            pltpu.sync_copy(asrc_hbm.at[isrc.at[m]], gs.at[pl.ds(m * 128, 128)])
            pltpu.sync_copy(adst_hbm.at[idst.at[m]], gd.at[pl.ds(m * 128, 128)])

        @pl.loop(0, BLK)
        def _edge(i):
            e = gs[i, :] + gd[i, :]
            e = jnp.maximum(e, e * 0.2)
            pbuf[i, :] = jnp.exp(e)

        pltpu.sync_copy(pbuf, p_hbm.at[pl.ds(ebase, BLK)])
        for m in range(4):
            pltpu.sync_copy(pbuf.at[pl.ds(m * 128, 128)],
                            dacc.at[idst.at[m]], add=True)

    plsc.subcore_barrier()
    pltpu.sync_copy(dacc.at[pl.ds(nbase, 512)],
                    dpart_hbm.at[c].at[pl.ds(nbase, 512)])
    pltpu.sync_copy(dacc.at[pl.ds(nbase + 512, NPT - 512)],
                    dpart_hbm.at[c].at[pl.ds(nbase + 512, NPT - 512)])


@functools.partial(
    pl.kernel, mesh=_mesh,
    out_type=jax.ShapeDtypeStruct((H, ROWS, 128), f32),
    scratch_types=[pltpu.VMEM((4, 128), i32),
                   pltpu.VMEM((BLK, 16), f32),
                   pltpu.VMEM((BLK, 16), f32),
                   pltpu.VMEM((BLK, 16), f32),
                   pltpu.VMEM((16, 4, 128), f32)])
def _phase_a2(dstP_hbm, p_hbm, dpart_hbm, at_hbm, idst, pbuf, d0, d1, tbuf):
    c = lax.axis_index("c")
    s = lax.axis_index("s")
    wid = c * NS + s
    row0 = wid * RPW
    lanes = lax.iota(i32, 16)

    @pl.loop(0, BPW)
    def _block(b):
        r = row0 + b * 4
        ebase = r * 128
        pltpu.sync_copy(dstP_hbm.at[pl.ds(r, 4)], idst)
        pltpu.sync_copy(p_hbm.at[pl.ds(ebase, BLK)], pbuf)
        for m in range(4):
            pltpu.sync_copy(dpart_hbm.at[0].at[idst.at[m]],
                            d0.at[pl.ds(m * 128, 128)])
            pltpu.sync_copy(dpart_hbm.at[1].at[idst.at[m]],
                            d1.at[pl.ds(m * 128, 128)])
        for m in range(4):
            mvec = jnp.full((16,), m, i32)

            @pl.loop(0, 128)
            def _edge(ii):
                i = m * 128 + ii
                a = pbuf[i, :] / (d0[i, :] + d1[i, :])
                plsc.store_scatter(tbuf, [lanes, mvec, jnp.full((16,), ii, i32)], a)

        pltpu.sync_copy(tbuf.at[pl.ds(0, 8)], at_hbm.at[:, pl.ds(r, 4), :])


@functools.partial(
    pl.kernel, mesh=_mesh,
    out_type=jax.ShapeDtypeStruct((NC, N, D), f32),
    scratch_types=[pltpu.VMEM((4, 128), i32),
                   pltpu.VMEM((4, 128), i32),
                   pltpu.VMEM((4, 128), f32),
                   pltpu.VMEM((BLK, D), f32),
                   pltpu.VMEM_SHARED((N, D), f32)])
def _phase_b(srcB_hbm, dstP_hbm, at_hbm, ht_hbm, outp_hbm,
             isrc, idst, abuf, gbuf, oacc):
    c = lax.axis_index("c")
    s = lax.axis_index("s")
    wid = c * NS + s
    nbase = s * NPT

    @pl.loop(0, BLK)
    def _zero(i):
        for k in range(8):
            gbuf[i, pl.ds(k * 16, 16)] = jnp.zeros((16,), f32)

    pltpu.sync_copy(gbuf.at[pl.ds(0, 512)], oacc.at[pl.ds(nbase, 512)])
    pltpu.sync_copy(gbuf.at[pl.ds(0, NPT - 512)],
                    oacc.at[pl.ds(nbase + 512, NPT - 512)])
    plsc.subcore_barrier()

    row0 = wid * RPW
    for j in range(H):
        @pl.loop(0, BPW)
        def _block(b):
            r = row0 + b * 4
            pltpu.sync_copy(srcB_hbm.at[pl.ds(r, 4)], isrc)
            pltpu.sync_copy(dstP_hbm.at[pl.ds(r, 4)], idst)
            pltpu.sync_copy(at_hbm.at[j, pl.ds(r, 4), :], abuf)
            for m in range(4):
                pltpu.sync_copy(ht_hbm.at[j].at[isrc.at[m]],
                                gbuf.at[pl.ds(m * 128, 128)])
            for m in range(4):
                @pl.loop(0, 128)
                def _edge(ii):
                    i = m * 128 + ii
                    av = jnp.full((16,), abuf[m, ii])
                    for k in range(8):
                        gbuf[i, pl.ds(k * 16, 16)] = gbuf[i, pl.ds(k * 16, 16)] * av

            for m in range(4):
                pltpu.sync_copy(gbuf.at[pl.ds(m * 128, 128)],
                                oacc.at[idst.at[m]], add=True)

    plsc.subcore_barrier()
    pltpu.sync_copy(oacc.at[pl.ds(nbase, 512)],
                    outp_hbm.at[c].at[pl.ds(nbase, 512)])
    pltpu.sync_copy(oacc.at[pl.ds(nbase + 512, NPT - 512)],
                    outp_hbm.at[c].at[pl.ds(nbase + 512, NPT - 512)])


def _logit_mat(a):
    m = jnp.einsum("hd,hk->hdk", a, jnp.eye(H, 16, dtype=a.dtype))
    return m.reshape(H * D, 16)


def kernel(x, edge_index, W1, a_src1, a_dst1, b1, W2, a_src2, a_dst2, b2):
    ei = edge_index.astype(i32)
    loop_idx = jnp.arange(N, dtype=i32)
    src = jnp.concatenate([ei[0], loop_idx])
    dst = jnp.concatenate([ei[1], loop_idx])
    npad = E_PAD - E_TOT
    srcA = jnp.concatenate([src, jnp.full((npad,), N, i32)]).reshape(ROWS, 128)
    srcB = jnp.concatenate([src, jnp.zeros((npad,), i32)]).reshape(ROWS, 128)
    dstP = jnp.concatenate([dst, jnp.zeros((npad,), i32)]).reshape(ROWS, 128)

    sentinel = jnp.full((16, 16), -1e30, f32)

    def layer(proj):
        ht, asr, adr = proj
        asrcF = jnp.concatenate([asr, sentinel], axis=0)
        p, dpart = _phase_a(srcA, dstP, asrcF, adr)
        at = _phase_a2(dstP, p, dpart)
        return _phase_b(srcB, dstP, at, ht)

    am1, bm1 = _logit_mat(a_src1), _logit_mat(a_dst1)
    am2, bm2 = _logit_mat(a_src2), _logit_mat(a_dst2)

    outp1 = layer(_tc_proj(x, W1, am1, bm1))
    outp2 = layer(_tc_mid(outp1, b1.reshape(1, D), W2, am2, bm2))
    return _tc_fin(outp2, b2.reshape(1, D))


# speedup vs baseline: 1.8788x; 1.8788x over previous
import dataclasses
import functools

import jax
import jax.numpy as jnp
from jax import lax
from jax.experimental import pallas as pl
from jax.experimental.pallas import tpu as pltpu
from jax.experimental.pallas import tpu_sc as plsc

N = 10000
NH = 5000
H = 8
D = 128
D_IN = 128
E_RAW = 320000
E_TOT = E_RAW + N
NC, NS = 2, 16
NW = NC * NS
RPW = 88
BPW = RPW // 8
ROWS = NW * RPW
E_PAD = ROWS * 128
CAP = RPW * 128
NREG = NW * 2
NA = 5120
NPT = NA // NS
DUMP = NH
BN = 1000

_mesh = plsc.VectorSubcoreMesh(core_axis_name="c", subcore_axis_name="s")
f32 = jnp.float32
i32 = jnp.int32

_sc_params = pltpu.CompilerParams()
if "needs_layout_passes" in pltpu.CompilerParams.__dataclass_fields__:
    _sc_params = dataclasses.replace(_sc_params, needs_layout_passes=False)


def _tc_proj_body(x_ref, w_ref, am_ref, bm_ref, ht_ref, as_ref, ad_ref):
    xb = x_ref[...]
    hb = jnp.dot(xb, w_ref[...], preferred_element_type=f32)
    for j in range(H):
        ht_ref[j] = hb[:, j * D:(j + 1) * D]
    as_ref[...] = jnp.dot(hb, am_ref[...], preferred_element_type=f32)
    ad_ref[...] = jnp.dot(hb, bm_ref[...], preferred_element_type=f32)


def _tc_proj(x, w, am, bm):
    return pl.pallas_call(
        _tc_proj_body,
        grid=(N // BN,),
        in_specs=[pl.BlockSpec((BN, D_IN), lambda i: (i, 0)),
                  pl.BlockSpec((D_IN, H * D), lambda i: (0, 0)),
                  pl.BlockSpec((H * D, 128), lambda i: (0, 0)),
                  pl.BlockSpec((H * D, 128), lambda i: (0, 0))],
        out_specs=[pl.BlockSpec((H, BN, D), lambda i: (0, i, 0)),
                   pl.BlockSpec((BN, 128), lambda i: (i, 0)),
                   pl.BlockSpec((BN, 128), lambda i: (i, 0))],
        out_shape=[jax.ShapeDtypeStruct((H, N, D), f32),
                   jax.ShapeDtypeStruct((N, 128), f32),
                   jax.ShapeDtypeStruct((N, 128), f32)],
    )(x, w, am, bm)


def _normalize(o_ref, dp_ref):
    rec = 1.0 / dp_ref[0]
    acc = o_ref[0, 0] * rec[:, 0:1]
    for j in range(1, H):
        acc = acc + o_ref[0, j] * rec[:, j:j + 1]
    return acc * (1.0 / H)


def _tc_mid_body(o_ref, dp_ref, b_ref, w_ref, am_ref, bm_ref,
                 ht_ref, as_ref, ad_ref):
    hin = jnp.maximum(_normalize(o_ref, dp_ref) + b_ref[...], 0.0)
    hb = jnp.dot(hin, w_ref[...], preferred_element_type=f32)
    for j in range(H):
        ht_ref[j] = hb[:, j * D:(j + 1) * D]
    as_ref[...] = jnp.dot(hb, am_ref[...], preferred_element_type=f32)
    ad_ref[...] = jnp.dot(hb, bm_ref[...], preferred_element_type=f32)


def _tc_mid(outp, dpart, b, w, am, bm):
    return pl.pallas_call(
        _tc_mid_body,
        grid=(N // BN,),
        in_specs=[pl.BlockSpec((1, H, BN, D), lambda i: (i // 5, 0, i % 5, 0)),
                  pl.BlockSpec((1, BN, 128), lambda i: (i // 5, i % 5, 0)),
                  pl.BlockSpec((1, D), lambda i: (0, 0)),
                  pl.BlockSpec((D, H * D), lambda i: (0, 0)),
                  pl.BlockSpec((H * D, 128), lambda i: (0, 0)),
                  pl.BlockSpec((H * D, 128), lambda i: (0, 0))],
        out_specs=[pl.BlockSpec((H, BN, D), lambda i: (0, i, 0)),
                   pl.BlockSpec((BN, 128), lambda i: (i, 0)),
                   pl.BlockSpec((BN, 128), lambda i: (i, 0))],
        out_shape=[jax.ShapeDtypeStruct((H, N, D), f32),
                   jax.ShapeDtypeStruct((N, 128), f32),
                   jax.ShapeDtypeStruct((N, 128), f32)],
    )(outp, dpart, b, w, am, bm)


def _tc_fin_body(o_ref, dp_ref, b_ref, out_ref):
    out_ref[...] = _normalize(o_ref, dp_ref) + b_ref[...]


def _tc_fin(outp, dpart, b):
    return pl.pallas_call(
        _tc_fin_body,
        grid=(N // BN,),
        in_specs=[pl.BlockSpec((1, H, BN, D), lambda i: (i // 5, 0, i % 5, 0)),
                  pl.BlockSpec((1, BN, 128), lambda i: (i // 5, i % 5, 0)),
                  pl.BlockSpec((1, D), lambda i: (0, 0))],
        out_specs=pl.BlockSpec((BN, D), lambda i: (i, 0)),
        out_shape=jax.ShapeDtypeStruct((N, D), f32),
    )(outp, dpart, b)


@functools.partial(
    pl.kernel, mesh=_mesh, compiler_params=_sc_params,
    out_type=(jax.ShapeDtypeStruct((NREG, RPW, 128), i32),
              jax.ShapeDtypeStruct((NREG, RPW, 128), i32),
              jax.ShapeDtypeStruct((NW, 8, 128), i32)),
    scratch_types=[pltpu.VMEM((8, 128), i32),
                   pltpu.VMEM((8, 128), i32),
                   pltpu.VMEM((89, 128), i32),
                   pltpu.VMEM((89, 128), i32),
                   pltpu.VMEM((89, 128), i32),
                   pltpu.VMEM((89, 128), i32),
                   pltpu.VMEM((8, 128), i32)])
def _sc_partition(srcP_hbm, dstP_hbm, bsrc_hbm, bdst_hbm, cnt_hbm,
                  isrc, idst, l0s, l0d, l1s, l1d, cbuf):
    c = lax.axis_index("c")
    s = lax.axis_index("s")
    wid = c * NS + s
    row0 = wid * RPW
    lanes = lax.iota(i32, 16)

    def chunk(b, carry):
        cnt0, cnt1 = carry
        r = row0 + b * 8
        pltpu.sync_copy(srcP_hbm.at[pl.ds(r, 8)], isrc)
        pltpu.sync_copy(dstP_hbm.at[pl.ds(r, 8)], idst)
        for m in range(8):
            for v in range(8):
                sv = isrc[m, pl.ds(v * 16, 16)]
                dv = idst[m, pl.ds(v * 16, 16)]
                m0 = dv < NH
                m1 = dv >= NH
                c0 = jnp.cumsum(m0.astype(i32))
                pos0 = cnt0 + c0 - 1
                plsc.store_scatter(l0s, [pos0 // 128, pos0 % 128], sv, mask=m0)
                plsc.store_scatter(l0d, [pos0 // 128, pos0 % 128], dv, mask=m0)
                n0 = c0[15]
                c1 = jnp.cumsum(m1.astype(i32))
                pos1 = cnt1 + c1 - 1
                plsc.store_scatter(l1s, [pos1 // 128, pos1 % 128], sv, mask=m1)
                plsc.store_scatter(l1d, [pos1 // 128, pos1 % 128], dv, mask=m1)
                cnt0 = cnt0 + n0
                cnt1 = cnt1 + (16 - n0)
        return cnt0, cnt1

    cnt0, cnt1 = lax.fori_loop(0, BPW, chunk, (jnp.int32(0), jnp.int32(0)))

    t0 = ((cnt0 + 1023) // 1024) * 1024
    t1 = ((cnt1 + 1023) // 1024) * 1024
    zs = jnp.zeros((16,), i32)
    d0pad = jnp.full((16,), NH, i32)
    d1pad = jnp.full((16,), N, i32)

    def pad0(k, cnt):
        pos = cnt + lanes
        plsc.store_scatter(l0s, [pos // 128, pos % 128], zs)
        plsc.store_scatter(l0d, [pos // 128, pos % 128], d0pad)
        return cnt + 16

    def pad1(k, cnt):
        pos = cnt + lanes
        plsc.store_scatter(l1s, [pos // 128, pos % 128], zs)
        plsc.store_scatter(l1d, [pos // 128, pos % 128], d1pad)
        return cnt + 16

    lax.fori_loop(0, (t0 - cnt0 + 15) // 16, pad0, cnt0)
    lax.fori_loop(0, (t1 - cnt1 + 15) // 16, pad1, cnt1)

    pltpu.sync_copy(l0s.at[pl.ds(0, RPW)], bsrc_hbm.at[wid * 2])
    pltpu.sync_copy(l0d.at[pl.ds(0, RPW)], bdst_hbm.at[wid * 2])
    pltpu.sync_copy(l1s.at[pl.ds(0, RPW)], bsrc_hbm.at[wid * 2 + 1])
    pltpu.sync_copy(l1d.at[pl.ds(0, RPW)], bdst_hbm.at[wid * 2 + 1])

    @pl.loop(0, 8)
    def _zc(i):
        for k in range(8):
            cbuf[i, pl.ds(k * 16, 16)] = jnp.zeros((16,), i32)

    cvec = jnp.where(lanes == 0, t0, jnp.where(lanes == 1, t1, 0))
    cbuf[0, pl.ds(0, 16)] = cvec.astype(i32)
    pltpu.sync_copy(cbuf, cnt_hbm.at[wid])


@functools.partial(
    pl.kernel, mesh=_mesh, compiler_params=_sc_params,
    out_type=(jax.ShapeDtypeStruct((H, NREG * RPW, 128), f32),
              jax.ShapeDtypeStruct((NC, NA, 128), f32),
              jax.ShapeDtypeStruct((NC, H, NA, D), f32)),
    scratch_types=[pltpu.VMEM((8, 128), i32),
                   pltpu.VMEM((8, 128), i32),
                   pltpu.VMEM((8, 128), i32),
                   pltpu.VMEM((8, 128), i32),
                   pltpu.VMEM((128, 128), f32),
                   pltpu.VMEM((128, 128), f32),
                   pltpu.VMEM((128, 128), f32),
                   pltpu.VMEM((16, 8, 128), f32),
                   pltpu.VMEM((8, 128), f32),
                   pltpu.VMEM((128, 128), f32),
                   pltpu.VMEM_SHARED((NA, 128), f32)])
def _sc_layer(bsrc_hbm, bdst_hbm, cnt_hbm, asrc_hbm, adst_hbm, ht_hbm,
              pt_hbm, dpart_hbm, outp_hbm,
              isrc, idst, lidx, cbuf, gs, gd, pw, tbuf, pb, gbuf, sacc):
    c = lax.axis_index("c")
    s = lax.axis_index("s")
    nbase = s * NPT
    lanes = lax.iota(i32, 16)
    coff = jnp.full((16,), c * NH, i32)

    def localize():
        @pl.loop(0, 8)
        def _lz(m):
            for v in range(8):
                lidx[m, pl.ds(v * 16, 16)] = (
                    idst[m, pl.ds(v * 16, 16)] - coff)

    def my_count(reg):
        pltpu.sync_copy(cnt_hbm.at[2 * s + reg], cbuf)
        av = cbuf[0, pl.ds(0, 16)]
        return jnp.where(c == 0, av[0], av[1])

    def zero_slice(buf):
        @pl.loop(0, 2)
        def _z(t):
            pltpu.sync_copy(buf, sacc.at[pl.ds(nbase + t * 128, 128)])
        pltpu.sync_copy(buf.at[pl.ds(0, 64)], sacc.at[pl.ds(nbase + 256, 64)])

    def flush_slice(dst):
        @pl.loop(0, 2)
        def _f(t):
            pltpu.sync_copy(sacc.at[pl.ds(nbase + t * 128, 128)],
                            dst.at[pl.ds(nbase + t * 128, 128)])
        pltpu.sync_copy(sacc.at[pl.ds(nbase + 256, 64)],
                        dst.at[pl.ds(nbase + 256, 64)])

    @pl.loop(0, 128)
    def _zpw(i):
        for k in range(8):
            pw[i, pl.ds(k * 16, 16)] = jnp.zeros((16,), f32)

    zero_slice(pw)
    plsc.subcore_barrier()

    for reg in range(2):
        p2 = (2 * s + reg) * 2 + c
        nb = my_count(reg) // 1024

        @pl.loop(0, nb)
        def _ablock(b):
            r = b * 8
            pltpu.sync_copy(bsrc_hbm.at[p2].at[pl.ds(r, 8)], isrc)
            pltpu.sync_copy(bdst_hbm.at[p2].at[pl.ds(r, 8)], idst)
            localize()
            for m in range(8):
                pltpu.sync_copy(asrc_hbm.at[isrc.at[m]], gs)
                pltpu.sync_copy(adst_hbm.at[idst.at[m]], gd)
                mvec = jnp.full((16,), m, i32)

                @pl.loop(0, 128)
                def _edge(ii):
                    e = gs[ii, pl.ds(0, 16)] + gd[ii, pl.ds(0, 16)]
                    e = jnp.maximum(e, e * 0.2)
                    p = jnp.exp(e)
                    pw[ii, pl.ds(0, 16)] = p
                    plsc.store_scatter(
                        tbuf, [lanes, mvec, jnp.full((16,), ii, i32)], p)

                pltpu.sync_copy(pw, sacc.at[lidx.at[m]], add=True)
            pltpu.sync_copy(tbuf.at[pl.ds(0, 8)],
                            pt_hbm.at[:, pl.ds(p2 * RPW + r, 8), :])

    plsc.subcore_barrier()
    flush_slice(dpart_hbm.at[c])

    @pl.loop(0, H)
    def _head(j):
        @pl.loop(0, 128)
        def _zg(i):
            for k in range(8):
                gbuf[i, pl.ds(k * 16, 16)] = jnp.zeros((16,), f32)

        zero_slice(gbuf)
        plsc.subcore_barrier()

        for reg in range(2):
            p2 = (2 * s + reg) * 2 + c
            nb = my_count(reg) // 1024

            @pl.loop(0, nb)
            def _bblock(b):
                r = b * 8
                pltpu.sync_copy(bsrc_hbm.at[p2].at[pl.ds(r, 8)], isrc)
                pltpu.sync_copy(bdst_hbm.at[p2].at[pl.ds(r, 8)], idst)
                localize()
                pltpu.sync_copy(pt_hbm.at[j, pl.ds(p2 * RPW + r, 8), :], pb)
                for m in range(8):
                    pltpu.sync_copy(ht_hbm.at[j].at[isrc.at[m]], gbuf)

                    @pl.loop(0, 8)
                    def _grp(g):
                        av16 = pb[m, pl.ds(g * 16, 16)]
                        for l in range(16):
                            i = g * 16 + l
                            av = jnp.full((16,), av16[l])
                            for k in range(8):
                                gbuf[i, pl.ds(k * 16, 16)] = (
                                    gbuf[i, pl.ds(k * 16, 16)] * av)

                    pltpu.sync_copy(gbuf, sacc.at[lidx.at[m]], add=True)

        plsc.subcore_barrier()
        flush_slice(outp_hbm.at[c].at[j])
        plsc.subcore_barrier()


def _logit_mat(a):
    m = jnp.einsum("hd,hk->hdk", a, jnp.eye(H, 128, dtype=a.dtype))
    return m.reshape(H * D, 128)


def kernel(x, edge_index, W1, a_src1, a_dst1, b1, W2, a_src2, a_dst2, b2):
    ei = edge_index.astype(i32)
    loop_idx = jnp.arange(N, dtype=i32)
    src = jnp.concatenate([ei[0], loop_idx])
    dst = jnp.concatenate([ei[1], loop_idx])
    npad = E_PAD - E_TOT
    srcP = jnp.concatenate([src, jnp.zeros((npad,), i32)]).reshape(ROWS, 128)
    dstP = jnp.concatenate([dst, jnp.full((npad,), N, i32)]).reshape(ROWS, 128)

    bsrc, bdst, cnts = _sc_partition(srcP, dstP)

    zrows = jnp.zeros((128, 128), f32)

    def layer(proj):
        ht, asr, adr = proj
        adrE = jnp.concatenate([adr, zrows], axis=0)
        _, dpart, outp = _sc_layer(bsrc, bdst, cnts, asr, adrE, ht)
        return outp, dpart

    am1, bm1 = _logit_mat(a_src1), _logit_mat(a_dst1)
    am2, bm2 = _logit_mat(a_src2), _logit_mat(a_dst2)

    outp1, dpart1 = layer(_tc_proj(x, W1, am1, bm1))
    outp2, dpart2 = layer(
        _tc_mid(outp1, dpart1, b1.reshape(1, D), W2, am2, bm2))
    return _tc_fin(outp2, dpart2, b2.reshape(1, D))
